# Initial kernel scaffold; baseline (speedup 1.0000x reference)
import functools

import jax
import jax.numpy as jnp
from jax import lax
from jax.experimental import pallas as pl
from jax.experimental.pallas import tpu as pltpu
from jax.experimental.pallas import tpu_sc as plsc

N = 10000
E = 320000
D = 128
H = 64
T = 10

NC = 2
NS = 16
NW = NC * NS
CH = 128
NCH = 80
EPW = NCH * CH
E_PAD = NW * EPW
N_PAD = 10016
ZR = N_PAD // NS
RPS = N // NS


def _mlp_a_body(x_ref, w1_ref, b1_ref, w2_ref, b2_ref, wh_ref, bh_ref,
                p_ref, h_ref):
    x = x_ref[...]
    t = jnp.tanh(jnp.dot(x, w1_ref[...], preferred_element_type=jnp.float32)
                 + b1_ref[...])
    filt = jax.nn.sigmoid(
        jnp.dot(t, w2_ref[...], preferred_element_type=jnp.float32)
        + b2_ref[...])
    h = jnp.maximum(
        jnp.dot(x, wh_ref[...], preferred_element_type=jnp.float32)
        + bh_ref[...], 0.0)
    h_ref[...] = h
    p_ref[...] = h * filt


_mlp_a = pl.pallas_call(
    _mlp_a_body,
    out_shape=(jax.ShapeDtypeStruct((N, H), jnp.float32),
               jax.ShapeDtypeStruct((N, H), jnp.float32)),
)


def _mlp_b_body(h_ref, agg_ref, wg_ref, bg_ref, wo_ref, bo_ref, out_ref):
    s = h_ref[...] + agg_ref[0] + agg_ref[1]
    h2 = jnp.maximum(
        jnp.dot(s, wg_ref[...], preferred_element_type=jnp.float32)
        + bg_ref[...], 0.0)
    out_ref[...] = (jnp.dot(h2, wo_ref[...], preferred_element_type=jnp.float32)
                    + bo_ref[...])


_mlp_b = pl.pallas_call(
    _mlp_b_body,
    out_shape=jax.ShapeDtypeStruct((N, T), jnp.float32),
)


@functools.partial(
    pl.kernel,
    out_type=jax.ShapeDtypeStruct((NC, N, H), jnp.float32),
    mesh=plsc.VectorSubcoreMesh(core_axis_name="c", subcore_axis_name="s"),
    scratch_types=[
        pltpu.VMEM((NCH, CH), jnp.int32),
        pltpu.VMEM((NCH, CH), jnp.int32),
        pltpu.VMEM((CH, H), jnp.float32),
        pltpu.VMEM_SHARED((N_PAD, H), jnp.float32),
        pltpu.SemaphoreType.DMA,
    ],
)
def _sc_agg(p_hbm, src_hbm, dst_hbm, zeros_hbm, out_hbm,
            src_v, dst_v, rows_v, acc_shared, sem):
    cid = lax.axis_index("c")
    sid = lax.axis_index("s")
    wid = cid * NS + sid
    pltpu.sync_copy(zeros_hbm.at[pl.ds(sid * ZR, ZR)],
                    acc_shared.at[pl.ds(sid * ZR, ZR)])
    pltpu.sync_copy(src_hbm.at[wid], src_v)
    pltpu.sync_copy(dst_hbm.at[wid], dst_v)
    plsc.subcore_barrier()

    @pl.loop(0, NCH)
    def _(j):
        pltpu.async_copy(p_hbm.at[src_v.at[j]], rows_v, sem).wait()
        pltpu.sync_copy(rows_v, acc_shared.at[dst_v.at[j]], add=True)

    plsc.subcore_barrier()
    pltpu.sync_copy(acc_shared.at[pl.ds(sid * RPS, RPS)],
                    out_hbm.at[cid, pl.ds(sid * RPS, RPS)])


def kernel(x, edge_index, W1, b1, W2, b2, Wh, bh, Wg, bg, Wo, bo):
    p, h = _mlp_a(x, W1, b1.reshape(1, H), W2, b2.reshape(1, H),
                  Wh, bh.reshape(1, H))
    pad = E_PAD - E
    src = jnp.concatenate([edge_index[0], jnp.zeros((pad,), jnp.int32)])
    trash = N + (jnp.arange(pad, dtype=jnp.int32) % (N_PAD - N))
    dst = jnp.concatenate([edge_index[1], trash])
    zeros = jnp.zeros((N_PAD, H), jnp.float32)
    agg = _sc_agg(p, src.reshape(NW, NCH, CH), dst.reshape(NW, NCH, CH), zeros)
    return _mlp_b(h, agg, Wg, bg.reshape(1, H), Wo, bo.reshape(1, H))


# speedup vs baseline: 6.1732x; 6.1732x over previous
import functools

import jax
import jax.numpy as jnp
from jax import lax
from jax.experimental import pallas as pl
from jax.experimental.pallas import tpu as pltpu
from jax.experimental.pallas import tpu_sc as plsc

N = 10000
E = 320000
D = 128
H = 64
T = 10

NC = 2
NS = 16
NW = NC * NS
CH = 128
NCH = 80
EPW = NCH * CH
E_PAD = NW * EPW
N_PAD = 10112
ZR = N_PAD // NS


def _mlp_a_body(x_ref, w1_ref, b1_ref, w2_ref, b2_ref, wh_ref, bh_ref,
                p_ref, h_ref):
    x = x_ref[...]
    t = jnp.tanh(jnp.dot(x, w1_ref[...], preferred_element_type=jnp.float32)
                 + b1_ref[...])
    filt = jax.nn.sigmoid(
        jnp.dot(t, w2_ref[...], preferred_element_type=jnp.float32)
        + b2_ref[...])
    h = jnp.maximum(
        jnp.dot(x, wh_ref[...], preferred_element_type=jnp.float32)
        + bh_ref[...], 0.0)
    h_ref[...] = h
    p_ref[...] = h * filt


_mlp_a = pl.pallas_call(
    _mlp_a_body,
    out_shape=(jax.ShapeDtypeStruct((N, H), jnp.float32),
               jax.ShapeDtypeStruct((N, H), jnp.float32)),
)


def _mlp_b_body(h_ref, agg_ref, wg_ref, bg_ref, wo_ref, bo_ref, out_ref):
    s = h_ref[...] + agg_ref[0, :N] + agg_ref[1, :N]
    h2 = jnp.maximum(
        jnp.dot(s, wg_ref[...], preferred_element_type=jnp.float32)
        + bg_ref[...], 0.0)
    out_ref[...] = (jnp.dot(h2, wo_ref[...], preferred_element_type=jnp.float32)
                    + bo_ref[...])


_mlp_b = pl.pallas_call(
    _mlp_b_body,
    out_shape=jax.ShapeDtypeStruct((N, T), jnp.float32),
)


@functools.partial(
    pl.kernel,
    out_type=jax.ShapeDtypeStruct((NC, N_PAD, H), jnp.float32),
    mesh=plsc.VectorSubcoreMesh(core_axis_name="c", subcore_axis_name="s"),
    scratch_types=[
        pltpu.VMEM((NCH, CH), jnp.int32),
        pltpu.VMEM((NCH, CH), jnp.int32),
        pltpu.VMEM((CH, H), jnp.float32),
        pltpu.VMEM_SHARED((N_PAD, H), jnp.float32),
        pltpu.SemaphoreType.DMA,
    ],
    compiler_params=pltpu.CompilerParams(use_tc_tiling_on_sc=False),
)
def _sc_agg(p_hbm, src_hbm, dst_hbm, zeros_hbm, out_hbm,
            src_v, dst_v, rows_v, acc_shared, sem):
    cid = lax.axis_index("c")
    sid = lax.axis_index("s")
    wid = cid * NS + sid
    pltpu.sync_copy(zeros_hbm.at[pl.ds(sid * ZR, ZR)],
                    acc_shared.at[pl.ds(sid * ZR, ZR)])
    pltpu.sync_copy(src_hbm.at[wid], src_v)
    pltpu.sync_copy(dst_hbm.at[wid], dst_v)
    plsc.subcore_barrier()

    @pl.loop(0, NCH)
    def _(j):
        pltpu.async_copy(p_hbm.at[src_v.at[j]], rows_v, sem).wait()
        pltpu.sync_copy(rows_v, acc_shared.at[dst_v.at[j]], add=True)

    plsc.subcore_barrier()
    pltpu.sync_copy(acc_shared.at[pl.ds(sid * ZR, ZR)],
                    out_hbm.at[cid, pl.ds(sid * ZR, ZR)])


def kernel(x, edge_index, W1, b1, W2, b2, Wh, bh, Wg, bg, Wo, bo):
    p, h = _mlp_a(x, W1, b1.reshape(1, H), W2, b2.reshape(1, H),
                  Wh, bh.reshape(1, H))
    pad = E_PAD - E
    src = jnp.concatenate([edge_index[0], jnp.zeros((pad,), jnp.int32)])
    trash = N + (jnp.arange(pad, dtype=jnp.int32) % (N_PAD - N))
    dst = jnp.concatenate([edge_index[1], trash])
    zeros = jnp.zeros((N_PAD, H), jnp.float32)
    agg = _sc_agg(p, src.reshape(NW, NCH, CH), dst.reshape(NW, NCH, CH), zeros)
    return _mlp_b(h, agg, Wg, bg.reshape(1, H), Wo, bo.reshape(1, T))


# speedup vs baseline: 7.0945x; 1.1492x over previous
import functools

import jax
import jax.numpy as jnp
from jax import lax
from jax.experimental import pallas as pl
from jax.experimental.pallas import tpu as pltpu
from jax.experimental.pallas import tpu_sc as plsc

N = 10000
E = 320000
D = 128
H = 64
T = 10

NC = 2
NS = 16
NW = NC * NS
CH = 128
NCH = 80
EPW = NCH * CH
E_PAD = NW * EPW
NBUF = 4
N_PAD = 10112
ZR = N_PAD // NS


def _mlp_a_body(x_ref, w1_ref, b1_ref, w2_ref, b2_ref, wh_ref, bh_ref,
                p_ref, h_ref):
    x = x_ref[...]
    t = jnp.tanh(jnp.dot(x, w1_ref[...], preferred_element_type=jnp.float32)
                 + b1_ref[...])
    filt = jax.nn.sigmoid(
        jnp.dot(t, w2_ref[...], preferred_element_type=jnp.float32)
        + b2_ref[...])
    h = jnp.maximum(
        jnp.dot(x, wh_ref[...], preferred_element_type=jnp.float32)
        + bh_ref[...], 0.0)
    h_ref[...] = h
    p_ref[...] = h * filt


_mlp_a = pl.pallas_call(
    _mlp_a_body,
    out_shape=(jax.ShapeDtypeStruct((N, H), jnp.float32),
               jax.ShapeDtypeStruct((N, H), jnp.float32)),
)


def _mlp_b_body(h_ref, agg_ref, wg_ref, bg_ref, wo_ref, bo_ref, out_ref):
    s = h_ref[...] + agg_ref[0, :N] + agg_ref[1, :N]
    h2 = jnp.maximum(
        jnp.dot(s, wg_ref[...], preferred_element_type=jnp.float32)
        + bg_ref[...], 0.0)
    out_ref[...] = (jnp.dot(h2, wo_ref[...], preferred_element_type=jnp.float32)
                    + bo_ref[...])


_mlp_b = pl.pallas_call(
    _mlp_b_body,
    out_shape=jax.ShapeDtypeStruct((N, T), jnp.float32),
)


@functools.partial(
    pl.kernel,
    out_type=jax.ShapeDtypeStruct((NC, N_PAD, H), jnp.float32),
    mesh=plsc.VectorSubcoreMesh(core_axis_name="c", subcore_axis_name="s"),
    scratch_types=[
        pltpu.VMEM((NCH, CH), jnp.int32),
        pltpu.VMEM((NCH, CH), jnp.int32),
        pltpu.VMEM((NBUF, CH, H), jnp.float32),
        pltpu.VMEM_SHARED((N_PAD, H), jnp.float32),
        pltpu.SemaphoreType.DMA((NBUF,)),
        pltpu.SemaphoreType.DMA((NBUF,)),
    ],
    compiler_params=pltpu.CompilerParams(use_tc_tiling_on_sc=False),
)
def _sc_agg(p_hbm, src_hbm, dst_hbm, zeros_hbm, out_hbm,
            src_v, dst_v, rows_v, acc_shared, gsem, ssem):
    cid = lax.axis_index("c")
    sid = lax.axis_index("s")
    wid = cid * NS + sid
    pltpu.sync_copy(zeros_hbm.at[pl.ds(sid * ZR, ZR)],
                    acc_shared.at[pl.ds(sid * ZR, ZR)])
    pltpu.sync_copy(src_hbm.at[wid], src_v)
    pltpu.sync_copy(dst_hbm.at[wid], dst_v)
    plsc.subcore_barrier()

    for b in range(NBUF):
        pltpu.async_copy(p_hbm.at[src_v.at[b]], rows_v.at[b], gsem.at[b])

    @pl.loop(0, NCH, step=NBUF)
    def _(j):
        for b in range(NBUF):
            pltpu.make_async_copy(p_hbm.at[src_v.at[j + b]], rows_v.at[b],
                                  gsem.at[b]).wait()
            pltpu.async_copy(rows_v.at[b], acc_shared.at[dst_v.at[j + b]],
                             ssem.at[b], add=True)
            pltpu.make_async_copy(rows_v.at[b], acc_shared.at[dst_v.at[j + b]],
                                  ssem.at[b]).wait()

            @pl.when(j + b + NBUF < NCH)
            def _():
                pltpu.async_copy(p_hbm.at[src_v.at[j + b + NBUF]],
                                 rows_v.at[b], gsem.at[b])

    plsc.subcore_barrier()
    pltpu.sync_copy(acc_shared.at[pl.ds(sid * ZR, ZR)],
                    out_hbm.at[cid, pl.ds(sid * ZR, ZR)])


def kernel(x, edge_index, W1, b1, W2, b2, Wh, bh, Wg, bg, Wo, bo):
    p, h = _mlp_a(x, W1, b1.reshape(1, H), W2, b2.reshape(1, H),
                  Wh, bh.reshape(1, H))
    pad = E_PAD - E
    src = jnp.concatenate([edge_index[0], jnp.zeros((pad,), jnp.int32)])
    trash = N + (jnp.arange(pad, dtype=jnp.int32) % (N_PAD - N))
    dst = jnp.concatenate([edge_index[1], trash])
    zeros = jnp.zeros((N_PAD, H), jnp.float32)
    agg = _sc_agg(p, src.reshape(NW, NCH, CH), dst.reshape(NW, NCH, CH), zeros)
    return _mlp_b(h, agg, Wg, bg.reshape(1, H), Wo, bo.reshape(1, T))
